# baseline (speedup 1.0000x reference)
import functools

import jax
import jax.numpy as jnp
from jax import lax
from jax.experimental import pallas as pl
from jax.experimental.pallas import tpu as pltpu
from jax.experimental.pallas import tpu_sc as plsc

_NC = 2
_NS = 16
_NW = _NC * _NS
_K = 4


@functools.partial(jax.jit, static_argnames=("b_per_w", "sp"))
def _sc_gather(table, idx_flat, *, b_per_w, sp):
    d = table.shape[1]
    rows_per_w = b_per_w * sp
    mesh = plsc.VectorSubcoreMesh(core_axis_name="c", subcore_axis_name="s")

    @functools.partial(
        pl.kernel,
        mesh=mesh,
        out_type=jax.ShapeDtypeStruct((b_per_w * _NW * sp, d), jnp.float32),
        scratch_types=[
            pltpu.VMEM((rows_per_w,), jnp.int32),
            *[pltpu.VMEM((sp, d), jnp.float32) for _ in range(_K)],
            pltpu.SemaphoreType.DMA,
        ],
    )
    def k(table_hbm, idx_hbm, out_hbm, idx_v, *rest):
        bufs, sem = rest[:_K], rest[_K]
        wid = lax.axis_index("s") * _NC + lax.axis_index("c")
        base = wid * rows_per_w
        pltpu.sync_copy(idx_hbm.at[pl.ds(base, rows_per_w)], idx_v)

        @pl.loop(0, b_per_w // _K)
        def _(i):
            copies = [
                pltpu.async_copy(
                    table_hbm.at[idx_v.at[pl.ds((i * _K + j) * sp, sp)]],
                    bufs[j],
                    sem,
                )
                for j in range(_K)
            ]
            for j in range(_K):
                copies[j].wait()
                pltpu.sync_copy(
                    bufs[j], out_hbm.at[pl.ds(base + (i * _K + j) * sp, sp)]
                )

    return k(table, idx_flat)


def kernel(inputs, kernel):
    table = kernel
    ids = inputs.astype(jnp.int32)
    n_rows, s = ids.shape
    assert n_rows % (_NW * _K) == 0
    b_per_w = n_rows // _NW
    sp = -(-s // 8) * 8
    if sp != s:
        ids = jnp.pad(ids, ((0, 0), (0, sp - s)))
    d = table.shape[1]
    out = _sc_gather(table, ids.reshape(-1), b_per_w=b_per_w, sp=sp)
    return out.reshape(n_rows, sp, d)[:, :s, :]
